# baseline (speedup 1.0000x reference)
import jax
import jax.numpy as jnp
from jax.experimental import pallas as pl
from jax.experimental.pallas import tpu as pltpu


def _make_kernel(nbb):
    def _mf_kernel(uid_ref,
                   user_hbm, label_hbm,
                   item_ref,
                   pid_ref, nid_ref,
                   user_out, pos_out, neg_out, com_out,
                   bl_buf, user_buf, sems):
        c = pl.program_id(0)
        kb = pl.program_id(1)
        num_items, dim = item_ref.shape
        nch = num_items // 128
        bt = bl_buf.shape[1] // nch
        blk = c * nbb + kb
        ph = kb % 2

        def issue(block, phase):
            base = block * bt
            for j in range(bt):
                u = uid_ref[base + j]
                pltpu.make_async_copy(
                    label_hbm.at[u],
                    bl_buf.at[phase, pl.ds(nch * j, nch), :],
                    sems.at[phase, j % 2]).start(priority=j % 2)
                pltpu.make_async_copy(
                    user_hbm.at[pl.ds(u, 1), :],
                    user_buf.at[phase, pl.ds(j, 1), :],
                    sems.at[phase, 2 + (j + 1) % 2]).start(priority=(j + 1) % 2)

        @pl.when(kb == 0)
        def _issue_first():
            issue(blk, 0)

        @pl.when(kb < nbb - 1)
        def _prefetch_next():
            issue(blk + 1, (kb + 1) % 2)

        item = item_ref[...]

        lane = jax.lax.broadcasted_iota(jnp.int32, (bt, num_items), 1)
        oh_pos = (pid_ref[...] == lane).astype(jnp.float32)
        oh_neg = (nid_ref[...] == lane).astype(jnp.float32)
        pos_out[...] = jnp.dot(oh_pos, item, preferred_element_type=jnp.float32)
        neg_out[...] = jnp.dot(oh_neg, item, preferred_element_type=jnp.float32)

        h = bt // 2
        for s in range(2):
            pltpu.make_async_copy(
                label_hbm.at[pl.ds(0, h)], bl_buf.at[0, pl.ds(0, nch * h), :],
                sems.at[ph, s]).wait()

        acc = jnp.zeros((bt, dim), jnp.float32)
        num = jnp.zeros((bt, 1), jnp.float32)
        for cch in range(nch):
            bl_c = bl_buf[ph, pl.Slice(cch, bt, nch), :]
            acc += jnp.dot(bl_c, item[cch * 128:(cch + 1) * 128, :],
                           preferred_element_type=jnp.float32)
            num += jnp.sum(bl_c, axis=1, keepdims=True)
        com_out[...] = acc / jnp.where(num > 0.0, num, 1.0)

        for s in range(2):
            pltpu.make_async_copy(
                user_hbm.at[pl.ds(0, h), :], user_buf.at[0, pl.ds(0, h), :],
                sems.at[ph, 2 + s]).wait()
        user_out[...] = user_buf[ph]

    return _mf_kernel


def kernel(user_id, pos_id, neg_id, user_table, item_table, train_label):
    bt = 256
    B = user_id.shape[0]
    num_users, dim = user_table.shape
    num_items = item_table.shape[0]

    nb = 2 * pl.cdiv(B, 2 * bt)
    nbb = nb // 2
    Bp = nb * bt
    pad = Bp - B

    uid = user_id.astype(jnp.int32)
    pid = pos_id.astype(jnp.int32)
    nid = neg_id.astype(jnp.int32)
    if pad:
        uid = jnp.pad(uid, (0, pad))
        pid = jnp.pad(pid, (0, pad))
        nid = jnp.pad(nid, (0, pad))
    pid2 = pid.reshape(Bp, 1)
    nid2 = nid.reshape(Bp, 1)

    grid_spec = pltpu.PrefetchScalarGridSpec(
        num_scalar_prefetch=1,
        grid=(2, nbb),
        in_specs=[
            pl.BlockSpec(memory_space=pl.ANY),
            pl.BlockSpec(memory_space=pl.ANY),
            pl.BlockSpec((num_items, dim), lambda c, kb, uid: (0, 0)),
            pl.BlockSpec((bt, 1), lambda c, kb, uid: (c * nbb + kb, 0)),
            pl.BlockSpec((bt, 1), lambda c, kb, uid: (c * nbb + kb, 0)),
        ],
        out_specs=[pl.BlockSpec((bt, dim),
                                lambda c, kb, uid: (c * nbb + kb, 0))] * 4,
        scratch_shapes=[
            pltpu.VMEM((2, bt * (num_items // 128), 128),
                       jnp.float32),
            pltpu.VMEM((2, bt, dim), jnp.float32),
            pltpu.SemaphoreType.DMA((2, 4)),
        ],
    )

    outs = pl.pallas_call(
        _make_kernel(nbb),
        out_shape=[jax.ShapeDtypeStruct((Bp, dim), jnp.float32)] * 4,
        grid_spec=grid_spec,
        compiler_params=pltpu.CompilerParams(
            dimension_semantics=("parallel", "arbitrary"),
            vmem_limit_bytes=60 * 1024 * 1024),
    )(uid,
      user_table.astype(jnp.float32),
      train_label.astype(jnp.float32).reshape(num_users, num_items // 128, 128),
      item_table.astype(jnp.float32),
      pid2, nid2)

    if pad:
        outs = [o[:B] for o in outs]
    return tuple(outs)


# speedup vs baseline: 4.4980x; 4.4980x over previous
import jax
import jax.numpy as jnp
from jax.experimental import pallas as pl
from jax.experimental.pallas import tpu as pltpu


def _make_kernel(nbb):
    def _mf_kernel(uid_ref,
                   user_hbm, label_hbm,
                   item_ref,
                   pid_ref, nid_ref,
                   user_out, pos_out, neg_out, com_out,
                   bl_buf, user_buf, sems):
        c = pl.program_id(0)
        kb = pl.program_id(1)
        _, bt, num_items = bl_buf.shape
        blk = c * nbb + kb
        ph = kb % 2

        def issue(block, phase):
            base = block * bt
            for j in range(bt):
                u = uid_ref[base + j]
                pltpu.make_async_copy(
                    label_hbm.at[pl.ds(u, 1), :],
                    bl_buf.at[phase, pl.ds(j, 1), :],
                    sems.at[phase, j % 2]).start(priority=j % 2)
                pltpu.make_async_copy(
                    user_hbm.at[pl.ds(u, 1), :],
                    user_buf.at[phase, pl.ds(j, 1), :],
                    sems.at[phase, 2 + (j + 1) % 2]).start(priority=(j + 1) % 2)

        @pl.when(kb == 0)
        def _issue_first():
            issue(blk, 0)

        @pl.when(kb < nbb - 1)
        def _prefetch_next():
            issue(blk + 1, (kb + 1) % 2)

        item = item_ref[...]

        lane = jax.lax.broadcasted_iota(jnp.int32, (bt, num_items), 1)
        oh_pos = (pid_ref[...] == lane).astype(jnp.float32)
        oh_neg = (nid_ref[...] == lane).astype(jnp.float32)
        pos_out[...] = jnp.dot(oh_pos, item, preferred_element_type=jnp.float32)
        neg_out[...] = jnp.dot(oh_neg, item, preferred_element_type=jnp.float32)

        h = bt // 2
        for s in range(2):
            pltpu.make_async_copy(
                label_hbm.at[pl.ds(0, h), :], bl_buf.at[0, pl.ds(0, h), :],
                sems.at[ph, s]).wait()

        bl = bl_buf[ph]
        acc = jnp.dot(bl, item, preferred_element_type=jnp.float32)
        num = jnp.sum(bl, axis=1, keepdims=True)
        com_out[...] = acc / jnp.where(num > 0.0, num, 1.0)

        for s in range(2):
            pltpu.make_async_copy(
                user_hbm.at[pl.ds(0, h), :], user_buf.at[0, pl.ds(0, h), :],
                sems.at[ph, 2 + s]).wait()
        user_out[...] = user_buf[ph]

    return _mf_kernel


def kernel(user_id, pos_id, neg_id, user_table, item_table, train_label):
    bt = 512
    B = user_id.shape[0]
    num_users, dim = user_table.shape
    num_items = item_table.shape[0]

    nb = 2 * pl.cdiv(B, 2 * bt)
    nbb = nb // 2
    Bp = nb * bt
    pad = Bp - B

    uid = user_id.astype(jnp.int32)
    pid = pos_id.astype(jnp.int32)
    nid = neg_id.astype(jnp.int32)
    if pad:
        uid = jnp.pad(uid, (0, pad))
        pid = jnp.pad(pid, (0, pad))
        nid = jnp.pad(nid, (0, pad))
    pid2 = pid.reshape(Bp, 1)
    nid2 = nid.reshape(Bp, 1)

    grid_spec = pltpu.PrefetchScalarGridSpec(
        num_scalar_prefetch=1,
        grid=(2, nbb),
        in_specs=[
            pl.BlockSpec(memory_space=pl.ANY),
            pl.BlockSpec(memory_space=pl.ANY),
            pl.BlockSpec((num_items, dim), lambda c, kb, uid: (0, 0)),
            pl.BlockSpec((bt, 1), lambda c, kb, uid: (c * nbb + kb, 0)),
            pl.BlockSpec((bt, 1), lambda c, kb, uid: (c * nbb + kb, 0)),
        ],
        out_specs=[pl.BlockSpec((bt, dim),
                                lambda c, kb, uid: (c * nbb + kb, 0))] * 4,
        scratch_shapes=[
            pltpu.VMEM((2, bt, num_items), jnp.float32),
            pltpu.VMEM((2, bt, dim), jnp.float32),
            pltpu.SemaphoreType.DMA((2, 4)),
        ],
    )

    outs = pl.pallas_call(
        _make_kernel(nbb),
        out_shape=[jax.ShapeDtypeStruct((Bp, dim), jnp.float32)] * 4,
        grid_spec=grid_spec,
        compiler_params=pltpu.CompilerParams(
            dimension_semantics=("parallel", "arbitrary"),
            vmem_limit_bytes=60 * 1024 * 1024),
    )(uid,
      user_table.astype(jnp.float32),
      train_label.astype(jnp.float32),
      item_table.astype(jnp.float32),
      pid2, nid2)

    if pad:
        outs = [o[:B] for o in outs]
    return tuple(outs)


# speedup vs baseline: 4.6840x; 1.0413x over previous
import jax
import jax.numpy as jnp
from jax.experimental import pallas as pl
from jax.experimental.pallas import tpu as pltpu


def _make_kernel(nbb):
    def _mf_kernel(uid_ref,
                   user_hbm, label_hbm,
                   item_ref,
                   pid_ref, nid_ref,
                   user_out, pos_out, neg_out, com_out,
                   bl_buf, user_buf, sems):
        c = pl.program_id(0)
        kb = pl.program_id(1)
        _, bt, num_items = bl_buf.shape
        blk = c * nbb + kb
        ph = kb % 2

        def issue(block, phase):
            base = block * bt
            for j in range(bt):
                u = uid_ref[base + j]
                pltpu.make_async_copy(
                    label_hbm.at[pl.ds(u, 1), :],
                    bl_buf.at[phase, pl.ds(j, 1), :],
                    sems.at[phase, j % 2]).start(priority=j % 2)
                pltpu.make_async_copy(
                    user_hbm.at[pl.ds(u, 1), :],
                    user_buf.at[phase, pl.ds(j, 1), :],
                    sems.at[phase, 2 + (j + 1) % 2]).start(priority=(j + 1) % 2)

        @pl.when(kb == 0)
        def _issue_first():
            issue(blk, 0)

        @pl.when(kb < nbb - 1)
        def _prefetch_next():
            issue(blk + 1, (kb + 1) % 2)

        item = item_ref[...]

        lane = jax.lax.broadcasted_iota(jnp.int32, (bt, num_items), 1)
        oh_pos = (pid_ref[...] == lane).astype(jnp.float32)
        oh_neg = (nid_ref[...] == lane).astype(jnp.float32)
        pos_out[...] = jnp.dot(oh_pos, item, preferred_element_type=jnp.float32)
        neg_out[...] = jnp.dot(oh_neg, item, preferred_element_type=jnp.float32)

        h = bt // 2
        for s in range(2):
            pltpu.make_async_copy(
                label_hbm.at[pl.ds(0, h), :], bl_buf.at[0, pl.ds(0, h), :],
                sems.at[ph, s]).wait()

        bl = bl_buf[ph]
        acc = jnp.dot(bl, item, preferred_element_type=jnp.float32)
        num = jnp.sum(bl, axis=1, keepdims=True)
        com_out[...] = acc / jnp.where(num > 0.0, num, 1.0)

        for s in range(2):
            pltpu.make_async_copy(
                user_hbm.at[pl.ds(0, h), :], user_buf.at[0, pl.ds(0, h), :],
                sems.at[ph, 2 + s]).wait()
        user_out[...] = user_buf[ph]

    return _mf_kernel


def kernel(user_id, pos_id, neg_id, user_table, item_table, train_label):
    bt = 1024
    B = user_id.shape[0]
    num_users, dim = user_table.shape
    num_items = item_table.shape[0]

    nb = 2 * pl.cdiv(B, 2 * bt)
    nbb = nb // 2
    Bp = nb * bt
    pad = Bp - B

    uid = user_id.astype(jnp.int32)
    pid = pos_id.astype(jnp.int32)
    nid = neg_id.astype(jnp.int32)
    if pad:
        uid = jnp.pad(uid, (0, pad))
        pid = jnp.pad(pid, (0, pad))
        nid = jnp.pad(nid, (0, pad))
    pid2 = pid.reshape(Bp, 1)
    nid2 = nid.reshape(Bp, 1)

    grid_spec = pltpu.PrefetchScalarGridSpec(
        num_scalar_prefetch=1,
        grid=(2, nbb),
        in_specs=[
            pl.BlockSpec(memory_space=pl.ANY),
            pl.BlockSpec(memory_space=pl.ANY),
            pl.BlockSpec((num_items, dim), lambda c, kb, uid: (0, 0)),
            pl.BlockSpec((bt, 1), lambda c, kb, uid: (c * nbb + kb, 0)),
            pl.BlockSpec((bt, 1), lambda c, kb, uid: (c * nbb + kb, 0)),
        ],
        out_specs=[pl.BlockSpec((bt, dim),
                                lambda c, kb, uid: (c * nbb + kb, 0))] * 4,
        scratch_shapes=[
            pltpu.VMEM((2, bt, num_items), jnp.float32),
            pltpu.VMEM((2, bt, dim), jnp.float32),
            pltpu.SemaphoreType.DMA((2, 4)),
        ],
    )

    outs = pl.pallas_call(
        _make_kernel(nbb),
        out_shape=[jax.ShapeDtypeStruct((Bp, dim), jnp.float32)] * 4,
        grid_spec=grid_spec,
        compiler_params=pltpu.CompilerParams(
            dimension_semantics=("parallel", "arbitrary"),
            vmem_limit_bytes=60 * 1024 * 1024),
    )(uid,
      user_table.astype(jnp.float32),
      train_label.astype(jnp.float32),
      item_table.astype(jnp.float32),
      pid2, nid2)

    if pad:
        outs = [o[:B] for o in outs]
    return tuple(outs)


# speedup vs baseline: 4.9933x; 1.0660x over previous
import jax
import jax.numpy as jnp
from jax.experimental import pallas as pl
from jax.experimental.pallas import tpu as pltpu


def _make_kernel(nbb):
    def _mf_kernel(uid_ref,
                   user_hbm, label_hbm,
                   item_ref,
                   pid_ref, nid_ref,
                   user_out, pos_out, neg_out, com_out,
                   bl_buf, user_buf, sems):
        c = pl.program_id(0)
        kb = pl.program_id(1)
        _, bt, num_items = bl_buf.shape
        blk = c * nbb + kb
        ph = kb % 2

        def issue(block, phase):
            base = block * bt
            for j in range(bt):
                u = uid_ref[base + j]
                pltpu.make_async_copy(
                    label_hbm.at[pl.ds(u, 1), :],
                    bl_buf.at[phase, pl.ds(j, 1), :],
                    sems.at[phase, j % 2]).start(priority=j % 2)
                pltpu.make_async_copy(
                    user_hbm.at[pl.ds(u, 1), :],
                    user_buf.at[phase, pl.ds(j, 1), :],
                    sems.at[phase, 2 + (j + 1) % 2]).start(priority=(j + 1) % 2)

        @pl.when(kb == 0)
        def _issue_first():
            issue(blk, 0)

        @pl.when(kb < nbb - 1)
        def _prefetch_next():
            issue(blk + 1, (kb + 1) % 2)

        item = item_ref[...]

        lane = jax.lax.broadcasted_iota(jnp.int32, (bt, num_items), 1)
        oh_pos = (pid_ref[...] == lane).astype(jnp.float32)
        oh_neg = (nid_ref[...] == lane).astype(jnp.float32)
        pos_out[...] = jnp.dot(oh_pos, item, preferred_element_type=jnp.float32)
        neg_out[...] = jnp.dot(oh_neg, item, preferred_element_type=jnp.float32)

        h = bt // 2
        for s in range(2):
            pltpu.make_async_copy(
                label_hbm.at[pl.ds(0, h), :], bl_buf.at[0, pl.ds(0, h), :],
                sems.at[ph, s]).wait()

        bl = bl_buf[ph]
        acc = jnp.dot(bl, item, preferred_element_type=jnp.float32)
        num = jnp.sum(bl, axis=1, keepdims=True)
        com_out[...] = acc / jnp.where(num > 0.0, num, 1.0)

        for s in range(2):
            pltpu.make_async_copy(
                user_hbm.at[pl.ds(0, h), :], user_buf.at[0, pl.ds(0, h), :],
                sems.at[ph, 2 + s]).wait()
        user_out[...] = user_buf[ph]

    return _mf_kernel


def kernel(user_id, pos_id, neg_id, user_table, item_table, train_label):
    bt = 2048
    B = user_id.shape[0]
    num_users, dim = user_table.shape
    num_items = item_table.shape[0]

    nb = 2 * pl.cdiv(B, 2 * bt)
    nbb = nb // 2
    Bp = nb * bt
    pad = Bp - B

    uid = user_id.astype(jnp.int32)
    pid = pos_id.astype(jnp.int32)
    nid = neg_id.astype(jnp.int32)
    if pad:
        uid = jnp.pad(uid, (0, pad))
        pid = jnp.pad(pid, (0, pad))
        nid = jnp.pad(nid, (0, pad))
    pid2 = pid.reshape(Bp, 1)
    nid2 = nid.reshape(Bp, 1)

    grid_spec = pltpu.PrefetchScalarGridSpec(
        num_scalar_prefetch=1,
        grid=(2, nbb),
        in_specs=[
            pl.BlockSpec(memory_space=pl.ANY),
            pl.BlockSpec(memory_space=pl.ANY),
            pl.BlockSpec((num_items, dim), lambda c, kb, uid: (0, 0)),
            pl.BlockSpec((bt, 1), lambda c, kb, uid: (c * nbb + kb, 0)),
            pl.BlockSpec((bt, 1), lambda c, kb, uid: (c * nbb + kb, 0)),
        ],
        out_specs=[pl.BlockSpec((bt, dim),
                                lambda c, kb, uid: (c * nbb + kb, 0))] * 4,
        scratch_shapes=[
            pltpu.VMEM((2, bt, num_items), jnp.float32),
            pltpu.VMEM((2, bt, dim), jnp.float32),
            pltpu.SemaphoreType.DMA((2, 4)),
        ],
    )

    outs = pl.pallas_call(
        _make_kernel(nbb),
        out_shape=[jax.ShapeDtypeStruct((Bp, dim), jnp.float32)] * 4,
        grid_spec=grid_spec,
        compiler_params=pltpu.CompilerParams(
            dimension_semantics=("parallel", "arbitrary"),
            vmem_limit_bytes=60 * 1024 * 1024),
    )(uid,
      user_table.astype(jnp.float32),
      train_label.astype(jnp.float32),
      item_table.astype(jnp.float32),
      pid2, nid2)

    if pad:
        outs = [o[:B] for o in outs]
    return tuple(outs)


# speedup vs baseline: 4.9941x; 1.0002x over previous
import jax
import jax.numpy as jnp
from jax.experimental import pallas as pl
from jax.experimental.pallas import tpu as pltpu


def _make_kernel(nbb):
    def _mf_kernel(uid_ref,
                   user_hbm, label_hbm,
                   item_ref,
                   pid_ref, nid_ref,
                   user_out, pos_out, neg_out, com_out,
                   bl_buf, user_buf, sems):
        c = pl.program_id(0)
        kb = pl.program_id(1)
        _, bt, num_items = bl_buf.shape
        blk = c * nbb + kb
        ph = kb % 2

        def issue(block, phase):
            base = block * bt
            for j in range(bt):
                u = uid_ref[base + j]
                pltpu.make_async_copy(
                    label_hbm.at[pl.ds(u, 1), :],
                    bl_buf.at[phase, pl.ds(j, 1), :],
                    sems.at[phase, j % 2]).start(priority=j % 2)
            for j in range(bt):
                u = uid_ref[base + j]
                pltpu.make_async_copy(
                    user_hbm.at[pl.ds(u, 1), :],
                    user_buf.at[phase, pl.ds(j, 1), :],
                    sems.at[phase, 2 + j % 2]).start(priority=j % 2)

        @pl.when(kb == 0)
        def _issue_first():
            issue(blk, 0)

        @pl.when(kb < nbb - 1)
        def _prefetch_next():
            issue(blk + 1, (kb + 1) % 2)

        item = item_ref[...]

        lane = jax.lax.broadcasted_iota(jnp.int32, (bt, num_items), 1)
        oh_pos = (pid_ref[...] == lane).astype(jnp.float32)
        oh_neg = (nid_ref[...] == lane).astype(jnp.float32)
        pos_out[...] = jnp.dot(oh_pos, item, preferred_element_type=jnp.float32)
        neg_out[...] = jnp.dot(oh_neg, item, preferred_element_type=jnp.float32)

        h = bt // 2
        for s in range(2):
            pltpu.make_async_copy(
                label_hbm.at[pl.ds(0, h), :], bl_buf.at[0, pl.ds(0, h), :],
                sems.at[ph, s]).wait()

        bl = bl_buf[ph]
        acc = jnp.dot(bl, item, preferred_element_type=jnp.float32)
        num = jnp.sum(bl, axis=1, keepdims=True)
        com_out[...] = acc / jnp.where(num > 0.0, num, 1.0)

        for s in range(2):
            pltpu.make_async_copy(
                user_hbm.at[pl.ds(0, h), :], user_buf.at[0, pl.ds(0, h), :],
                sems.at[ph, 2 + s]).wait()
        user_out[...] = user_buf[ph]

    return _mf_kernel


def kernel(user_id, pos_id, neg_id, user_table, item_table, train_label):
    bt = 2048
    B = user_id.shape[0]
    num_users, dim = user_table.shape
    num_items = item_table.shape[0]

    nb = 2 * pl.cdiv(B, 2 * bt)
    nbb = nb // 2
    Bp = nb * bt
    pad = Bp - B

    uid = user_id.astype(jnp.int32)
    pid = pos_id.astype(jnp.int32)
    nid = neg_id.astype(jnp.int32)
    if pad:
        uid = jnp.pad(uid, (0, pad))
        pid = jnp.pad(pid, (0, pad))
        nid = jnp.pad(nid, (0, pad))
    pid2 = pid.reshape(Bp, 1)
    nid2 = nid.reshape(Bp, 1)

    grid_spec = pltpu.PrefetchScalarGridSpec(
        num_scalar_prefetch=1,
        grid=(2, nbb),
        in_specs=[
            pl.BlockSpec(memory_space=pl.ANY),
            pl.BlockSpec(memory_space=pl.ANY),
            pl.BlockSpec((num_items, dim), lambda c, kb, uid: (0, 0)),
            pl.BlockSpec((bt, 1), lambda c, kb, uid: (c * nbb + kb, 0)),
            pl.BlockSpec((bt, 1), lambda c, kb, uid: (c * nbb + kb, 0)),
        ],
        out_specs=[pl.BlockSpec((bt, dim),
                                lambda c, kb, uid: (c * nbb + kb, 0))] * 4,
        scratch_shapes=[
            pltpu.VMEM((2, bt, num_items), jnp.float32),
            pltpu.VMEM((2, bt, dim), jnp.float32),
            pltpu.SemaphoreType.DMA((2, 4)),
        ],
    )

    outs = pl.pallas_call(
        _make_kernel(nbb),
        out_shape=[jax.ShapeDtypeStruct((Bp, dim), jnp.float32)] * 4,
        grid_spec=grid_spec,
        compiler_params=pltpu.CompilerParams(
            dimension_semantics=("parallel", "arbitrary"),
            vmem_limit_bytes=60 * 1024 * 1024),
    )(uid,
      user_table.astype(jnp.float32),
      train_label.astype(jnp.float32),
      item_table.astype(jnp.float32),
      pid2, nid2)

    if pad:
        outs = [o[:B] for o in outs]
    return tuple(outs)
